# baseline (speedup 1.0000x reference)
import functools
import math

import jax
import jax.numpy as jnp
from jax import lax
from jax.experimental import pallas as pl
from jax.experimental.pallas import tpu as pltpu
from jax.experimental.pallas import tpu_sc as plsc

H = 64
_NC, _NS, _L = 2, 16, 16


def _silu(x):
    return x / (1.0 + jnp.exp(-x))


@functools.partial(jax.jit, static_argnames=("N", "E"))
def _sc_scatter(h, ef, src, dst, *, N, E):
    HALF = N // _NC
    ROWS = 25088
    TROWS = ROWS // _NS
    ZCH = 56
    ZIT = TROWS // ZCH
    WR = 1568
    WR_LAST = HALF - (_NS - 1) * WR
    K = 80
    EPT = E // _NS
    NCH = EPT // K
    NCHP = NCH - 1
    NP = NCHP // 2

    mesh = plsc.VectorSubcoreMesh(core_axis_name="c", subcore_axis_name="s",
                                  num_cores=_NC, num_subcores=_NS)

    @functools.partial(
        pl.kernel,
        mesh=mesh,
        out_type=jax.ShapeDtypeStruct((N, H), jnp.float32),
        compiler_params=pltpu.CompilerParams(use_tc_tiling_on_sc=False),
        scratch_types=[
            pltpu.VMEM((ZCH, H), jnp.float32),
            [pltpu.VMEM((K,), jnp.int32) for _ in range(2)],
            [pltpu.VMEM((K,), jnp.int32) for _ in range(2)],
            [pltpu.VMEM((K,), jnp.int32) for _ in range(2)],
            [pltpu.VMEM((K, H), jnp.float32) for _ in range(2)],
            [pltpu.VMEM((K // 2, 2 * H), jnp.float32) for _ in range(2)],
            pltpu.VMEM_SHARED((ROWS, H), jnp.float32),
            [pltpu.SemaphoreType.DMA for _ in range(2)],
            [pltpu.SemaphoreType.DMA for _ in range(2)],
            [pltpu.SemaphoreType.DMA for _ in range(2)],
            [pltpu.SemaphoreType.DMA for _ in range(2)],
        ],
    )
    def scat(h_hbm, ef_hbm, src_hbm, dst_hbm, out_hbm, zbuf, srcs, dsts, idxs,
             hrs, efs, aggs, semi, semg, seme, sems):
        c = lax.axis_index("c")
        s = lax.axis_index("s")

        zero16 = jnp.zeros((_L,), jnp.float32)

        def zfill(i, carry):
            r = i // (H // _L)
            q = i % (H // _L)
            zbuf[r, pl.ds(q * _L, _L)] = zero16
            return carry

        lax.fori_loop(0, ZCH * (H // _L), zfill, 0)

        def zcopy(i, carry):
            pltpu.sync_copy(zbuf, aggs.at[pl.ds(s * TROWS + i * ZCH, ZCH)])
            return carry

        lax.fori_loop(0, ZIT, zcopy, 0)
        plsc.subcore_barrier()

        base_row = c * HALF
        ebase = s * EPT

        def issue_idx(ch, b):
            eb = ebase + ch * K
            pltpu.async_copy(src_hbm.at[pl.ds(eb, K)], srcs[b], semi[b])
            pltpu.async_copy(dst_hbm.at[pl.ds(eb, K)], dsts[b], semi[b])

        def drain_idx(b):
            pltpu.make_async_copy(src_hbm.at[pl.ds(0, K)], srcs[b],
                                  semi[b]).wait()
            pltpu.make_async_copy(dst_hbm.at[pl.ds(0, K)], dsts[b],
                                  semi[b]).wait()

        def compute_idx(dref, iref, n):
            def ixb(j, cr):
                d = dref[pl.ds(j * _L, _L)]
                loc = d - base_row
                ok = (loc >= 0) & (loc < HALF)
                iref[pl.ds(j * _L, _L)] = jnp.where(ok, loc, HALF)
                return cr
            lax.fori_loop(0, n // _L, ixb, 0)

        def compute_msg(href, eref, n):
            def row(rr, cr):
                for half in (0, 1):
                    r = 2 * rr + half
                    for q in range(H // _L):
                        sl = pl.ds(q * _L, _L)
                        esl = pl.ds(half * H + q * _L, _L)
                        href[r, sl] = jnp.maximum(href[r, sl] + eref[rr, esl],
                                                  0.0)
                return cr
            lax.fori_loop(0, n // 2, row, 0)

        issue_idx(0, 0)
        drain_idx(0)
        pltpu.async_copy(h_hbm.at[srcs[0]], hrs[0], semg[0])
        pltpu.async_copy(ef_hbm.at[pl.ds(ebase // 2, K // 2)], efs[0], seme[0])
        issue_idx(1, 1)

        def pair(j, carry):
            for b in (0, 1):
                ch = 2 * j + b
                nb = 1 - b

                compute_idx(dsts[b], idxs[b], K)

                @pl.when(ch + 1 < NCHP)
                def _():
                    @pl.when(ch >= 1)
                    def _():
                        pltpu.make_async_copy(hrs[nb], aggs.at[pl.ds(0, K)],
                                              sems[nb]).wait()
                    drain_idx(nb)
                    pltpu.async_copy(h_hbm.at[srcs[nb]], hrs[nb], semg[nb])
                    pltpu.async_copy(
                        ef_hbm.at[pl.ds((ebase + (ch + 1) * K) // 2, K // 2)],
                        efs[nb], seme[nb])

                pltpu.make_async_copy(h_hbm.at[pl.ds(0, K)], hrs[b],
                                      semg[b]).wait()
                pltpu.make_async_copy(ef_hbm.at[pl.ds(0, K // 2)], efs[b],
                                      seme[b]).wait()

                @pl.when(ch + 2 < NCHP)
                def _():
                    issue_idx(ch + 2, b)

                compute_msg(hrs[b], efs[b], K)
                pltpu.async_copy(hrs[b], aggs.at[idxs[b]], sems[b], add=True)
            return carry

        lax.fori_loop(0, NP, pair, 0)
        pltpu.make_async_copy(hrs[0], aggs.at[pl.ds(0, K)], sems[0]).wait()
        pltpu.make_async_copy(hrs[1], aggs.at[pl.ds(0, K)], sems[1]).wait()

        tb = ebase + NCHP * K
        pltpu.sync_copy(src_hbm.at[pl.ds(tb, K)], srcs[0])
        cp = pltpu.async_copy(h_hbm.at[srcs[0]], hrs[0], semg[0])
        pltpu.sync_copy(dst_hbm.at[pl.ds(tb, K)], dsts[0])
        pltpu.sync_copy(ef_hbm.at[pl.ds(tb // 2, K // 2)], efs[0])
        compute_idx(dsts[0], idxs[0], K)
        cp.wait()
        compute_msg(hrs[0], efs[0], K)
        pltpu.sync_copy(hrs[0], aggs.at[idxs[0]], add=True)
        plsc.subcore_barrier()

        @pl.when(s < _NS - 1)
        def _():
            pltpu.sync_copy(
                aggs.at[pl.ds(s * WR, WR)],
                out_hbm.at[pl.ds(c * HALF + s * WR, WR)],
            )

        @pl.when(s == _NS - 1)
        def _():
            pltpu.sync_copy(
                aggs.at[pl.ds((_NS - 1) * WR, WR_LAST)],
                out_hbm.at[pl.ds(c * HALF + (_NS - 1) * WR, WR_LAST)],
            )

    return scat(h, ef, src, dst)


_NBLK = 1000
_EBLK = 6400


def _dot(a, b):
    return jnp.dot(a, b, preferred_element_type=jnp.float32)


def _node_encoder(x_in, batch2, te, w1, b1, w2, b2, tw, tb):
    N = x_in.shape[0]
    grid = N // _NBLK

    def body(xb, bb, teb, w1b, b1b, w2b, b2b, twb, tbb, out):
        pre = _dot(xb[...], w1b[...]) + b1b[...]
        hcur = _dot(_silu(pre), w2b[...]) + b2b[...]
        tf = _dot(_silu(teb[...]), twb[...]) + tbb[...]
        bbv = bb[...]
        for k in range(16):
            hcur += jnp.where(bbv == k, 1.0, 0.0) * tf[k:k + 1, :]
        out[...] = hcur

    full = lambda shape: pl.BlockSpec(shape, lambda i: (0, 0))
    return pl.pallas_call(
        body,
        grid=(grid,),
        in_specs=[
            pl.BlockSpec((_NBLK, 7), lambda i: (i, 0)),
            pl.BlockSpec((_NBLK, 1), lambda i: (i, 0)),
            full((16, 32)),
            full((7, H)), full((1, H)),
            full((H, H)), full((1, H)),
            full((32, H)), full((1, H)),
        ],
        out_specs=pl.BlockSpec((_NBLK, H), lambda i: (i, 0)),
        out_shape=jax.ShapeDtypeStruct((N, H), jnp.float32),
    )(x_in, batch2, te, w1, b1, w2, b2, tw, tb)


def _edge_mlp(eaP, w1P, b1P, w2d, b2P):
    E2 = eaP.shape[1]
    grid = E2 // (_EBLK // 2)

    def body(eb, w1b, b1b, w2b, b2b, out):
        pre = lax.dot_general(eb[...], w1b[...], (((0,), (0,)), ((), ())),
                              preferred_element_type=jnp.float32)
        hid = _silu(pre + b1b[...])
        out[...] = _dot(hid, w2b[...]) + b2b[...]

    full = lambda shape: pl.BlockSpec(shape, lambda i: (0, 0))
    return pl.pallas_call(
        body,
        grid=(grid,),
        in_specs=[
            pl.BlockSpec((4, _EBLK // 2), lambda i: (0, i)),
            full((4, 2 * H)), full((1, 2 * H)),
            full((2 * H, 2 * H)), full((1, 2 * H)),
        ],
        out_specs=pl.BlockSpec((_EBLK // 2, 2 * H), lambda i: (i, 0)),
        out_shape=jax.ShapeDtypeStruct((E2, 2 * H), jnp.float32),
    )(eaP, w1P, b1P, w2d, b2P)


def _layer_mlp(h, aggp, w1, b1, w2, b2):
    N = h.shape[0]
    grid = N // _NBLK

    def body(hb, ab, w1b, b1b, w2b, b2b, out):
        z = hb[...] + ab[...]
        hid = _silu(_dot(z, w1b[...]) + b1b[...])
        out[...] = _silu(_dot(hid, w2b[...]) + b2b[...])

    full = lambda shape: pl.BlockSpec(shape, lambda i: (0, 0))
    return pl.pallas_call(
        body,
        grid=(grid,),
        in_specs=[
            pl.BlockSpec((_NBLK, H), lambda i: (i, 0)),
            pl.BlockSpec((_NBLK, H), lambda i: (i, 0)),
            full((H, H)), full((1, H)), full((H, H)), full((1, H)),
        ],
        out_specs=pl.BlockSpec((_NBLK, H), lambda i: (i, 0)),
        out_shape=jax.ShapeDtypeStruct((N, H), jnp.float32),
    )(h, aggp, w1, b1, w2, b2)


def _final_mlp(h, aggp, w1, b1, w2, b2, fw1, fb1, fw2, fb2):
    N = h.shape[0]
    grid = N // _NBLK

    def body(hb, ab, w1b, b1b, w2b, b2b, fw1b, fb1b, fw2b, fb2b, out):
        z = hb[...] + ab[...]
        hid = _silu(_dot(z, w1b[...]) + b1b[...])
        h3 = _silu(_dot(hid, w2b[...]) + b2b[...])
        fh = _silu(_dot(h3, fw1b[...]) + fb1b[...])
        out[...] = _dot(fh, fw2b[...]) + fb2b[...]

    full = lambda shape: pl.BlockSpec(shape, lambda i: (0, 0))
    return pl.pallas_call(
        body,
        grid=(grid,),
        in_specs=[
            pl.BlockSpec((_NBLK, H), lambda i: (i, 0)),
            pl.BlockSpec((_NBLK, H), lambda i: (i, 0)),
            full((H, H)), full((1, H)), full((H, H)), full((1, H)),
            full((H, H)), full((1, H)), full((H, 1)), full((1, 1)),
        ],
        out_specs=pl.BlockSpec((_NBLK, 1), lambda i: (i, 0)),
        out_shape=jax.ShapeDtypeStruct((N, 1), jnp.float32),
    )(h, aggp, w1, b1, w2, b2, fw1, fb1, fw2, fb2)


def kernel(x_t, t, condition, edge_index, edge_attr, batch, params):
    N = x_t.shape[0]
    E = edge_index.shape[1]
    p = params

    half = 16
    freq = jnp.exp(jnp.arange(half, dtype=jnp.float32)
                   * (-math.log(10000.0) / (half - 1)))
    ang = t[:, None].astype(jnp.float32) * freq[None, :]
    te = jnp.concatenate([jnp.sin(ang), jnp.cos(ang)], axis=-1)

    r1 = lambda v: v.reshape(1, -1)
    batch2 = batch.reshape(N, 1)

    x_in = jnp.concatenate([x_t, condition], axis=-1)
    h = _node_encoder(
        x_in, batch2, te,
        p['node_w1'], r1(p['node_b1']),
        p['node_w2'], r1(p['node_b2']),
        p['time_w'], r1(p['time_b']),
    )
    eaP = edge_attr.T.reshape(2, E // 2, 2).transpose(0, 2, 1).reshape(4, E // 2)
    zr = jnp.zeros((1, H), jnp.float32)
    w1 = p['edge_w1']
    w1P = jnp.concatenate([
        jnp.concatenate([w1[0:1], zr], axis=1),
        jnp.concatenate([zr, w1[0:1]], axis=1),
        jnp.concatenate([w1[1:2], zr], axis=1),
        jnp.concatenate([zr, w1[1:2]], axis=1),
    ], axis=0)
    zH = jnp.zeros((H, H), jnp.float32)
    w2 = p['edge_w2']
    w2d = jnp.concatenate([
        jnp.concatenate([w2, zH], axis=1),
        jnp.concatenate([zH, w2], axis=1),
    ], axis=0)
    dup = lambda v: jnp.concatenate([v, v]).reshape(1, 2 * H)
    ef = _edge_mlp(eaP, w1P, dup(p['edge_b1']), w2d, dup(p['edge_b2']))

    src, dst = edge_index[0], edge_index[1]
    for name in ('c1', 'c2'):
        aggp = _sc_scatter(h, ef, src, dst, N=N, E=E)
        h = _layer_mlp(h, aggp, p[name + '_w1'], r1(p[name + '_b1']),
                       p[name + '_w2'], r1(p[name + '_b2']))

    aggp = _sc_scatter(h, ef, src, dst, N=N, E=E)
    return _final_mlp(h, aggp, p['c3_w1'], r1(p['c3_b1']),
                      p['c3_w2'], r1(p['c3_b2']),
                      p['f_w1'], r1(p['f_b1']),
                      p['f_w2'], jnp.full((1, 1), p['f_b2'][0]))


# speedup vs baseline: 1.0003x; 1.0003x over previous
import functools
import math

import jax
import jax.numpy as jnp
from jax import lax
from jax.experimental import pallas as pl
from jax.experimental.pallas import tpu as pltpu
from jax.experimental.pallas import tpu_sc as plsc

H = 64
_NC, _NS, _L = 2, 16, 16


def _silu(x):
    return x / (1.0 + jnp.exp(-x))


@functools.partial(jax.jit, static_argnames=("N", "E"))
def _sc_scatter(h, ef, src, dst, *, N, E):
    HALF = N // _NC
    ROWS = 25088
    TROWS = ROWS // _NS
    ZCH = 56
    ZIT = TROWS // ZCH
    WR = 1568
    WR_LAST = HALF - (_NS - 1) * WR
    K = 80
    EPT = E // _NS
    NCH = EPT // K
    NCHP = NCH - 1
    NP = NCHP // 2

    mesh = plsc.VectorSubcoreMesh(core_axis_name="c", subcore_axis_name="s",
                                  num_cores=_NC, num_subcores=_NS)

    @functools.partial(
        pl.kernel,
        mesh=mesh,
        out_type=jax.ShapeDtypeStruct((N, H), jnp.float32),
        compiler_params=pltpu.CompilerParams(use_tc_tiling_on_sc=False),
        scratch_types=[
            pltpu.VMEM((ZCH, H), jnp.float32),
            [pltpu.VMEM((K,), jnp.int32) for _ in range(2)],
            [pltpu.VMEM((K,), jnp.int32) for _ in range(2)],
            [pltpu.VMEM((K,), jnp.int32) for _ in range(2)],
            [pltpu.VMEM((K, H), jnp.float32) for _ in range(2)],
            [pltpu.VMEM((K // 2, 2 * H), jnp.float32) for _ in range(2)],
            pltpu.VMEM_SHARED((ROWS, H), jnp.float32),
            [pltpu.SemaphoreType.DMA for _ in range(2)],
            [pltpu.SemaphoreType.DMA for _ in range(2)],
            [pltpu.SemaphoreType.DMA for _ in range(2)],
            [pltpu.SemaphoreType.DMA for _ in range(2)],
        ],
    )
    def scat(h_hbm, ef_hbm, src_hbm, dst_hbm, out_hbm, zbuf, srcs, dsts, idxs,
             hrs, efs, aggs, semi, semg, seme, sems):
        c = lax.axis_index("c")
        s = lax.axis_index("s")

        zero16 = jnp.zeros((_L,), jnp.float32)

        def zfill(i, carry):
            r = i // (H // _L)
            q = i % (H // _L)
            zbuf[r, pl.ds(q * _L, _L)] = zero16
            return carry

        lax.fori_loop(0, ZCH * (H // _L), zfill, 0)

        def zcopy(i, carry):
            pltpu.sync_copy(zbuf, aggs.at[pl.ds(s * TROWS + i * ZCH, ZCH)])
            return carry

        lax.fori_loop(0, ZIT, zcopy, 0)
        plsc.subcore_barrier()

        base_row = c * HALF
        ebase = s * EPT

        def issue_idx(ch, b):
            eb = ebase + ch * K
            pltpu.async_copy(src_hbm.at[pl.ds(eb, K)], srcs[b], semi[b])
            pltpu.async_copy(dst_hbm.at[pl.ds(eb, K)], dsts[b], semi[b])

        def drain_idx(b):
            pltpu.make_async_copy(src_hbm.at[pl.ds(0, K)], srcs[b],
                                  semi[b]).wait()
            pltpu.make_async_copy(dst_hbm.at[pl.ds(0, K)], dsts[b],
                                  semi[b]).wait()

        def compute_idx(dref, iref, n):
            def ixb(j, cr):
                d = dref[pl.ds(j * _L, _L)]
                loc = d - base_row
                ok = (loc >= 0) & (loc < HALF)
                iref[pl.ds(j * _L, _L)] = jnp.where(ok, loc, HALF)
                return cr
            lax.fori_loop(0, n // _L, ixb, 0)

        def compute_msg(href, eref, n):
            def row(rr, cr):
                for half in (0, 1):
                    r = 2 * rr + half
                    for q in range(H // _L):
                        sl = pl.ds(q * _L, _L)
                        esl = pl.ds(half * H + q * _L, _L)
                        href[r, sl] = jnp.maximum(href[r, sl] + eref[rr, esl],
                                                  0.0)
                return cr
            lax.fori_loop(0, n // 2, row, 0)

        issue_idx(0, 0)
        drain_idx(0)
        pltpu.async_copy(h_hbm.at[srcs[0]], hrs[0], semg[0])
        pltpu.async_copy(ef_hbm.at[pl.ds(ebase // 2, K // 2)], efs[0], seme[0])
        issue_idx(1, 1)

        def pair(j, carry):
            for b in (0, 1):
                ch = 2 * j + b
                nb = 1 - b

                compute_idx(dsts[b], idxs[b], K)

                @pl.when(ch + 1 < NCHP)
                def _():
                    @pl.when(ch >= 1)
                    def _():
                        pltpu.make_async_copy(hrs[nb], aggs.at[pl.ds(0, K)],
                                              sems[nb]).wait()
                    drain_idx(nb)
                    pltpu.async_copy(h_hbm.at[srcs[nb]], hrs[nb], semg[nb])
                    pltpu.async_copy(
                        ef_hbm.at[pl.ds((ebase + (ch + 1) * K) // 2, K // 2)],
                        efs[nb], seme[nb])

                pltpu.make_async_copy(h_hbm.at[pl.ds(0, K)], hrs[b],
                                      semg[b]).wait()
                pltpu.make_async_copy(ef_hbm.at[pl.ds(0, K // 2)], efs[b],
                                      seme[b]).wait()

                @pl.when(ch + 2 < NCHP)
                def _():
                    issue_idx(ch + 2, b)

                compute_msg(hrs[b], efs[b], K)
                pltpu.async_copy(hrs[b], aggs.at[idxs[b]], sems[b], add=True)
            return carry

        lax.fori_loop(0, NP, pair, 0)
        pltpu.make_async_copy(hrs[0], aggs.at[pl.ds(0, K)], sems[0]).wait()
        pltpu.make_async_copy(hrs[1], aggs.at[pl.ds(0, K)], sems[1]).wait()

        tb = ebase + NCHP * K
        pltpu.sync_copy(src_hbm.at[pl.ds(tb, K)], srcs[0])
        cp = pltpu.async_copy(h_hbm.at[srcs[0]], hrs[0], semg[0])
        pltpu.sync_copy(dst_hbm.at[pl.ds(tb, K)], dsts[0])
        pltpu.sync_copy(ef_hbm.at[pl.ds(tb // 2, K // 2)], efs[0])
        compute_idx(dsts[0], idxs[0], K)
        cp.wait()
        compute_msg(hrs[0], efs[0], K)
        pltpu.sync_copy(hrs[0], aggs.at[idxs[0]], add=True)
        plsc.subcore_barrier()

        @pl.when(s < _NS - 1)
        def _():
            pltpu.sync_copy(
                aggs.at[pl.ds(s * WR, WR)],
                out_hbm.at[pl.ds(c * HALF + s * WR, WR)],
            )

        @pl.when(s == _NS - 1)
        def _():
            pltpu.sync_copy(
                aggs.at[pl.ds((_NS - 1) * WR, WR_LAST)],
                out_hbm.at[pl.ds(c * HALF + (_NS - 1) * WR, WR_LAST)],
            )

    return scat(h, ef, src, dst)


_NBLK = 1000
_EBLK = 6400


def _dot(a, b):
    return jnp.dot(a, b, preferred_element_type=jnp.float32)


def _node_encoder(x_in, batch2, te, w1, b1, w2, b2, tw, tb):
    N = x_in.shape[0]
    grid = N // _NBLK

    def body(xb, bb, teb, w1b, b1b, w2b, b2b, twb, tbb, out):
        pre = _dot(xb[...], w1b[...]) + b1b[...]
        hcur = _dot(_silu(pre), w2b[...]) + b2b[...]
        tf = _dot(_silu(teb[...]), twb[...]) + tbb[...]
        ids = lax.broadcasted_iota(jnp.int32, (_NBLK, 16), 1)
        oh = (bb[...] == ids).astype(jnp.float32)
        out[...] = hcur + jnp.dot(oh, tf, precision=lax.Precision.HIGHEST,
                                  preferred_element_type=jnp.float32)

    full = lambda shape: pl.BlockSpec(shape, lambda i: (0, 0))
    return pl.pallas_call(
        body,
        grid=(grid,),
        in_specs=[
            pl.BlockSpec((_NBLK, 7), lambda i: (i, 0)),
            pl.BlockSpec((_NBLK, 1), lambda i: (i, 0)),
            full((16, 32)),
            full((7, H)), full((1, H)),
            full((H, H)), full((1, H)),
            full((32, H)), full((1, H)),
        ],
        out_specs=pl.BlockSpec((_NBLK, H), lambda i: (i, 0)),
        out_shape=jax.ShapeDtypeStruct((N, H), jnp.float32),
    )(x_in, batch2, te, w1, b1, w2, b2, tw, tb)


def _edge_mlp(eaP, w1P, b1P, w2d, b2P):
    E2 = eaP.shape[1]
    grid = E2 // (_EBLK // 2)

    def body(eb, w1b, b1b, w2b, b2b, out):
        pre = lax.dot_general(eb[...], w1b[...], (((0,), (0,)), ((), ())),
                              preferred_element_type=jnp.float32)
        hid = _silu(pre + b1b[...])
        out[...] = _dot(hid, w2b[...]) + b2b[...]

    full = lambda shape: pl.BlockSpec(shape, lambda i: (0, 0))
    return pl.pallas_call(
        body,
        grid=(grid,),
        in_specs=[
            pl.BlockSpec((4, _EBLK // 2), lambda i: (0, i)),
            full((4, 2 * H)), full((1, 2 * H)),
            full((2 * H, 2 * H)), full((1, 2 * H)),
        ],
        out_specs=pl.BlockSpec((_EBLK // 2, 2 * H), lambda i: (i, 0)),
        out_shape=jax.ShapeDtypeStruct((E2, 2 * H), jnp.float32),
    )(eaP, w1P, b1P, w2d, b2P)


def _layer_mlp(h, aggp, w1, b1, w2, b2):
    N = h.shape[0]
    grid = N // _NBLK

    def body(hb, ab, w1b, b1b, w2b, b2b, out):
        z = hb[...] + ab[...]
        hid = _silu(_dot(z, w1b[...]) + b1b[...])
        out[...] = _silu(_dot(hid, w2b[...]) + b2b[...])

    full = lambda shape: pl.BlockSpec(shape, lambda i: (0, 0))
    return pl.pallas_call(
        body,
        grid=(grid,),
        in_specs=[
            pl.BlockSpec((_NBLK, H), lambda i: (i, 0)),
            pl.BlockSpec((_NBLK, H), lambda i: (i, 0)),
            full((H, H)), full((1, H)), full((H, H)), full((1, H)),
        ],
        out_specs=pl.BlockSpec((_NBLK, H), lambda i: (i, 0)),
        out_shape=jax.ShapeDtypeStruct((N, H), jnp.float32),
    )(h, aggp, w1, b1, w2, b2)


def _final_mlp(h, aggp, w1, b1, w2, b2, fw1, fb1, fw2, fb2):
    N = h.shape[0]
    grid = N // _NBLK

    def body(hb, ab, w1b, b1b, w2b, b2b, fw1b, fb1b, fw2b, fb2b, out):
        z = hb[...] + ab[...]
        hid = _silu(_dot(z, w1b[...]) + b1b[...])
        h3 = _silu(_dot(hid, w2b[...]) + b2b[...])
        fh = _silu(_dot(h3, fw1b[...]) + fb1b[...])
        out[...] = _dot(fh, fw2b[...]) + fb2b[...]

    full = lambda shape: pl.BlockSpec(shape, lambda i: (0, 0))
    return pl.pallas_call(
        body,
        grid=(grid,),
        in_specs=[
            pl.BlockSpec((_NBLK, H), lambda i: (i, 0)),
            pl.BlockSpec((_NBLK, H), lambda i: (i, 0)),
            full((H, H)), full((1, H)), full((H, H)), full((1, H)),
            full((H, H)), full((1, H)), full((H, 1)), full((1, 1)),
        ],
        out_specs=pl.BlockSpec((_NBLK, 1), lambda i: (i, 0)),
        out_shape=jax.ShapeDtypeStruct((N, 1), jnp.float32),
    )(h, aggp, w1, b1, w2, b2, fw1, fb1, fw2, fb2)


def kernel(x_t, t, condition, edge_index, edge_attr, batch, params):
    N = x_t.shape[0]
    E = edge_index.shape[1]
    p = params

    half = 16
    freq = jnp.exp(jnp.arange(half, dtype=jnp.float32)
                   * (-math.log(10000.0) / (half - 1)))
    ang = t[:, None].astype(jnp.float32) * freq[None, :]
    te = jnp.concatenate([jnp.sin(ang), jnp.cos(ang)], axis=-1)

    r1 = lambda v: v.reshape(1, -1)
    batch2 = batch.reshape(N, 1)

    x_in = jnp.concatenate([x_t, condition], axis=-1)
    h = _node_encoder(
        x_in, batch2, te,
        p['node_w1'], r1(p['node_b1']),
        p['node_w2'], r1(p['node_b2']),
        p['time_w'], r1(p['time_b']),
    )
    eaP = edge_attr.T.reshape(2, E // 2, 2).transpose(0, 2, 1).reshape(4, E // 2)
    zr = jnp.zeros((1, H), jnp.float32)
    w1 = p['edge_w1']
    w1P = jnp.concatenate([
        jnp.concatenate([w1[0:1], zr], axis=1),
        jnp.concatenate([zr, w1[0:1]], axis=1),
        jnp.concatenate([w1[1:2], zr], axis=1),
        jnp.concatenate([zr, w1[1:2]], axis=1),
    ], axis=0)
    zH = jnp.zeros((H, H), jnp.float32)
    w2 = p['edge_w2']
    w2d = jnp.concatenate([
        jnp.concatenate([w2, zH], axis=1),
        jnp.concatenate([zH, w2], axis=1),
    ], axis=0)
    dup = lambda v: jnp.concatenate([v, v]).reshape(1, 2 * H)
    ef = _edge_mlp(eaP, w1P, dup(p['edge_b1']), w2d, dup(p['edge_b2']))

    src, dst = edge_index[0], edge_index[1]
    for name in ('c1', 'c2'):
        aggp = _sc_scatter(h, ef, src, dst, N=N, E=E)
        h = _layer_mlp(h, aggp, p[name + '_w1'], r1(p[name + '_b1']),
                       p[name + '_w2'], r1(p[name + '_b2']))

    aggp = _sc_scatter(h, ef, src, dst, N=N, E=E)
    return _final_mlp(h, aggp, p['c3_w1'], r1(p['c3_b1']),
                      p['c3_w2'], r1(p['c3_b2']),
                      p['f_w1'], r1(p['f_b1']),
                      p['f_w2'], jnp.full((1, 1), p['f_b2'][0]))


# speedup vs baseline: 1.0869x; 1.0865x over previous
import functools
import math

import jax
import jax.numpy as jnp
from jax import lax
from jax.experimental import pallas as pl
from jax.experimental.pallas import tpu as pltpu
from jax.experimental.pallas import tpu_sc as plsc

H = 64
_NC, _NS, _L = 2, 16, 16


def _silu(x):
    return x / (1.0 + jnp.exp(-x))


@functools.partial(jax.jit, static_argnames=("N", "E"))
def _sc_scatter(h, ef, src, dst, *, N, E):
    HALF = N // _NC
    ROWS = 25088
    TROWS = ROWS // _NS
    ZCH = 56
    ZIT = TROWS // ZCH
    WR = 1568
    WR_LAST = HALF - (_NS - 1) * WR
    K = 80
    EPT = E // _NS
    NCH = EPT // K
    NCHP = NCH - 1
    NP = NCHP // 2

    mesh = plsc.VectorSubcoreMesh(core_axis_name="c", subcore_axis_name="s",
                                  num_cores=_NC, num_subcores=_NS)

    @functools.partial(
        pl.kernel,
        mesh=mesh,
        out_type=jax.ShapeDtypeStruct((N, H), jnp.float32),
        compiler_params=pltpu.CompilerParams(use_tc_tiling_on_sc=False),
        scratch_types=[
            pltpu.VMEM((ZCH, H), jnp.float32),
            [pltpu.VMEM((K,), jnp.int32) for _ in range(2)],
            [pltpu.VMEM((K,), jnp.int32) for _ in range(2)],
            [pltpu.VMEM((K,), jnp.int32) for _ in range(2)],
            [pltpu.VMEM((K, H), jnp.float32) for _ in range(2)],
            [pltpu.VMEM((K // 2, 2 * H), jnp.float32) for _ in range(2)],
            pltpu.VMEM_SHARED((ROWS, H), jnp.float32),
            [pltpu.SemaphoreType.DMA for _ in range(2)],
            [pltpu.SemaphoreType.DMA for _ in range(2)],
            [pltpu.SemaphoreType.DMA for _ in range(2)],
            [pltpu.SemaphoreType.DMA for _ in range(2)],
        ],
    )
    def scat(h_hbm, ef_hbm, src_hbm, dst_hbm, out_hbm, zbuf, srcs, dsts, idxs,
             hrs, efs, aggs, semi, semg, seme, sems):
        c = lax.axis_index("c")
        s = lax.axis_index("s")

        zero16 = jnp.zeros((_L,), jnp.float32)

        def zfill(i, carry):
            r = i // (H // _L)
            q = i % (H // _L)
            zbuf[r, pl.ds(q * _L, _L)] = zero16
            return carry

        lax.fori_loop(0, ZCH * (H // _L), zfill, 0)

        def zcopy(i, carry):
            pltpu.sync_copy(zbuf, aggs.at[pl.ds(s * TROWS + i * ZCH, ZCH)])
            return carry

        lax.fori_loop(0, ZIT, zcopy, 0)
        plsc.subcore_barrier()

        base_row = c * HALF
        ebase = s * EPT

        def issue_idx(ch, b):
            eb = ebase + ch * K
            pltpu.async_copy(src_hbm.at[pl.ds(eb, K)], srcs[b], semi[b])
            pltpu.async_copy(dst_hbm.at[pl.ds(eb, K)], dsts[b], semi[b])

        def drain_idx(b):
            pltpu.make_async_copy(src_hbm.at[pl.ds(0, K)], srcs[b],
                                  semi[b]).wait()
            pltpu.make_async_copy(dst_hbm.at[pl.ds(0, K)], dsts[b],
                                  semi[b]).wait()

        def compute_idx(dref, iref, n):
            def ixb(j, cr):
                d = dref[pl.ds(j * _L, _L)]
                loc = d - base_row
                ok = (loc >= 0) & (loc < HALF)
                iref[pl.ds(j * _L, _L)] = jnp.where(ok, loc, HALF)
                return cr
            lax.fori_loop(0, n // _L, ixb, 0)

        def compute_msg(href, eref, n):
            def row(rr, cr):
                for half in (0, 1):
                    r = 2 * rr + half
                    for q in range(H // _L):
                        sl = pl.ds(q * _L, _L)
                        esl = pl.ds(half * H + q * _L, _L)
                        href[r, sl] = jnp.maximum(href[r, sl] + eref[rr, esl],
                                                  0.0)
                return cr
            lax.fori_loop(0, n // 2, row, 0)

        issue_idx(0, 0)
        drain_idx(0)
        pltpu.async_copy(h_hbm.at[srcs[0]], hrs[0], semg[0])
        pltpu.async_copy(ef_hbm.at[pl.ds(ebase // 2, K // 2)], efs[0], seme[0])
        issue_idx(1, 1)

        def pair(j, carry):
            for b in (0, 1):
                ch = 2 * j + b
                nb = 1 - b

                compute_idx(dsts[b], idxs[b], K)

                @pl.when(ch + 1 < NCHP)
                def _():
                    @pl.when(ch >= 1)
                    def _():
                        pltpu.make_async_copy(hrs[nb], aggs.at[pl.ds(0, K)],
                                              sems[nb]).wait()
                    drain_idx(nb)
                    pltpu.async_copy(h_hbm.at[srcs[nb]], hrs[nb], semg[nb])
                    pltpu.async_copy(
                        ef_hbm.at[pl.ds((ebase + (ch + 1) * K) // 2, K // 2)],
                        efs[nb], seme[nb])

                pltpu.make_async_copy(h_hbm.at[pl.ds(0, K)], hrs[b],
                                      semg[b]).wait()
                pltpu.make_async_copy(ef_hbm.at[pl.ds(0, K // 2)], efs[b],
                                      seme[b]).wait()

                @pl.when(ch + 2 < NCHP)
                def _():
                    issue_idx(ch + 2, b)

                compute_msg(hrs[b], efs[b], K)
                pltpu.async_copy(hrs[b], aggs.at[idxs[b]], sems[b], add=True)
            return carry

        lax.fori_loop(0, NP, pair, 0)
        pltpu.make_async_copy(hrs[0], aggs.at[pl.ds(0, K)], sems[0]).wait()
        pltpu.make_async_copy(hrs[1], aggs.at[pl.ds(0, K)], sems[1]).wait()

        tb = ebase + NCHP * K
        pltpu.sync_copy(src_hbm.at[pl.ds(tb, K)], srcs[0])
        cp = pltpu.async_copy(h_hbm.at[srcs[0]], hrs[0], semg[0])
        pltpu.sync_copy(dst_hbm.at[pl.ds(tb, K)], dsts[0])
        pltpu.sync_copy(ef_hbm.at[pl.ds(tb // 2, K // 2)], efs[0])
        compute_idx(dsts[0], idxs[0], K)
        cp.wait()
        compute_msg(hrs[0], efs[0], K)
        pltpu.sync_copy(hrs[0], aggs.at[idxs[0]], add=True)
        plsc.subcore_barrier()

        @pl.when(s < _NS - 1)
        def _():
            pltpu.sync_copy(
                aggs.at[pl.ds(s * WR, WR)],
                out_hbm.at[pl.ds(c * HALF + s * WR, WR)],
            )

        @pl.when(s == _NS - 1)
        def _():
            pltpu.sync_copy(
                aggs.at[pl.ds((_NS - 1) * WR, WR_LAST)],
                out_hbm.at[pl.ds(c * HALF + (_NS - 1) * WR, WR_LAST)],
            )

    return scat(h, ef, src, dst)


_NBLK = 1000
_EBLK = 6400


def _dot(a, b):
    return jnp.dot(a, b, preferred_element_type=jnp.float32)


def _node_encoder(x_in, batch2, te, w1, b1, w2, b2, tw, tb):
    N = x_in.shape[0]
    grid = N // _NBLK

    def body(xb, bb, teb, w1b, b1b, w2b, b2b, twb, tbb, out):
        pre = _dot(xb[...], w1b[...]) + b1b[...]
        hcur = _dot(_silu(pre), w2b[...]) + b2b[...]
        tf = _dot(_silu(teb[...]), twb[...]) + tbb[...]
        ids = lax.broadcasted_iota(jnp.int32, (_NBLK, 16), 1)
        oh = (bb[...] == ids).astype(jnp.float32)
        out[...] = hcur + jnp.dot(oh, tf, precision=lax.Precision.HIGHEST,
                                  preferred_element_type=jnp.float32)

    full = lambda shape: pl.BlockSpec(shape, lambda i: (0, 0))
    return pl.pallas_call(
        body,
        grid=(grid,),
        in_specs=[
            pl.BlockSpec((_NBLK, 7), lambda i: (i, 0)),
            pl.BlockSpec((_NBLK, 1), lambda i: (i, 0)),
            full((16, 32)),
            full((7, H)), full((1, H)),
            full((H, H)), full((1, H)),
            full((32, H)), full((1, H)),
        ],
        out_specs=pl.BlockSpec((_NBLK, H), lambda i: (i, 0)),
        out_shape=jax.ShapeDtypeStruct((N, H), jnp.float32),
    )(x_in, batch2, te, w1, b1, w2, b2, tw, tb)


def _edge_mlp(eaT, w1, b1, w2, b2):
    E = eaT.shape[1]
    grid = E // _EBLK

    def body(eb, w1b, b1b, w2b, b2b, out):
        pre = lax.dot_general(eb[...], w1b[...], (((0,), (0,)), ((), ())),
                              preferred_element_type=jnp.float32)
        hid = _silu(pre + b1b[...])
        out[...] = _dot(hid, w2b[...]) + b2b[...]

    full = lambda shape: pl.BlockSpec(shape, lambda i: (0, 0))
    return pl.pallas_call(
        body,
        grid=(grid,),
        in_specs=[
            pl.BlockSpec((2, _EBLK), lambda i: (0, i)),
            full((2, H)), full((1, H)), full((H, H)), full((1, H)),
        ],
        out_specs=pl.BlockSpec((_EBLK, H), lambda i: (i, 0)),
        out_shape=jax.ShapeDtypeStruct((E, H), jnp.float32),
    )(eaT, w1, b1, w2, b2)


def _layer_mlp(h, aggp, w1, b1, w2, b2):
    N = h.shape[0]
    grid = N // _NBLK

    def body(hb, ab, w1b, b1b, w2b, b2b, out):
        z = hb[...] + ab[...]
        hid = _silu(_dot(z, w1b[...]) + b1b[...])
        out[...] = _silu(_dot(hid, w2b[...]) + b2b[...])

    full = lambda shape: pl.BlockSpec(shape, lambda i: (0, 0))
    return pl.pallas_call(
        body,
        grid=(grid,),
        in_specs=[
            pl.BlockSpec((_NBLK, H), lambda i: (i, 0)),
            pl.BlockSpec((_NBLK, H), lambda i: (i, 0)),
            full((H, H)), full((1, H)), full((H, H)), full((1, H)),
        ],
        out_specs=pl.BlockSpec((_NBLK, H), lambda i: (i, 0)),
        out_shape=jax.ShapeDtypeStruct((N, H), jnp.float32),
    )(h, aggp, w1, b1, w2, b2)


def _final_mlp(h, aggp, w1, b1, w2, b2, fw1, fb1, fw2, fb2):
    N = h.shape[0]
    grid = N // _NBLK

    def body(hb, ab, w1b, b1b, w2b, b2b, fw1b, fb1b, fw2b, fb2b, out):
        z = hb[...] + ab[...]
        hid = _silu(_dot(z, w1b[...]) + b1b[...])
        h3 = _silu(_dot(hid, w2b[...]) + b2b[...])
        fh = _silu(_dot(h3, fw1b[...]) + fb1b[...])
        out[...] = _dot(fh, fw2b[...]) + fb2b[...]

    full = lambda shape: pl.BlockSpec(shape, lambda i: (0, 0))
    return pl.pallas_call(
        body,
        grid=(grid,),
        in_specs=[
            pl.BlockSpec((_NBLK, H), lambda i: (i, 0)),
            pl.BlockSpec((_NBLK, H), lambda i: (i, 0)),
            full((H, H)), full((1, H)), full((H, H)), full((1, H)),
            full((H, H)), full((1, H)), full((H, 1)), full((1, 1)),
        ],
        out_specs=pl.BlockSpec((_NBLK, 1), lambda i: (i, 0)),
        out_shape=jax.ShapeDtypeStruct((N, 1), jnp.float32),
    )(h, aggp, w1, b1, w2, b2, fw1, fb1, fw2, fb2)


def kernel(x_t, t, condition, edge_index, edge_attr, batch, params):
    N = x_t.shape[0]
    E = edge_index.shape[1]
    p = params

    half = 16
    freq = jnp.exp(jnp.arange(half, dtype=jnp.float32)
                   * (-math.log(10000.0) / (half - 1)))
    ang = t[:, None].astype(jnp.float32) * freq[None, :]
    te = jnp.concatenate([jnp.sin(ang), jnp.cos(ang)], axis=-1)

    r1 = lambda v: v.reshape(1, -1)
    batch2 = batch.reshape(N, 1)

    x_in = jnp.concatenate([x_t, condition], axis=-1)
    h = _node_encoder(
        x_in, batch2, te,
        p['node_w1'], r1(p['node_b1']),
        p['node_w2'], r1(p['node_b2']),
        p['time_w'], r1(p['time_b']),
    )
    ef = _edge_mlp(edge_attr.T, p['edge_w1'], r1(p['edge_b1']),
                   p['edge_w2'], r1(p['edge_b2'])).reshape(E // 2, 2 * H)

    src, dst = edge_index[0], edge_index[1]
    for name in ('c1', 'c2'):
        aggp = _sc_scatter(h, ef, src, dst, N=N, E=E)
        h = _layer_mlp(h, aggp, p[name + '_w1'], r1(p[name + '_b1']),
                       p[name + '_w2'], r1(p[name + '_b2']))

    aggp = _sc_scatter(h, ef, src, dst, N=N, E=E)
    return _final_mlp(h, aggp, p['c3_w1'], r1(p['c3_b1']),
                      p['c3_w2'], r1(p['c3_b2']),
                      p['f_w1'], r1(p['f_b1']),
                      p['f_w2'], jnp.full((1, 1), p['f_b2'][0]))


# speedup vs baseline: 1.0979x; 1.0101x over previous
import functools
import math

import jax
import jax.numpy as jnp
from jax import lax
from jax.experimental import pallas as pl
from jax.experimental.pallas import tpu as pltpu
from jax.experimental.pallas import tpu_sc as plsc

H = 64
_NC, _NS, _L = 2, 16, 16


def _silu(x):
    return x / (1.0 + jnp.exp(-x))


@functools.partial(jax.jit, static_argnames=("N", "E"))
def _sc_scatter(h, ef, ei1d, *, N, E):
    HALF = N // _NC
    ROWS = 25088
    TROWS = ROWS // _NS
    ZCH = 56
    ZIT = TROWS // ZCH
    WR = 1568
    WR_LAST = HALF - (_NS - 1) * WR
    K = 80
    EPT = E // _NS
    NCH = EPT // K
    NCHP = NCH - 1
    NP = NCHP // 2

    mesh = plsc.VectorSubcoreMesh(core_axis_name="c", subcore_axis_name="s",
                                  num_cores=_NC, num_subcores=_NS)

    @functools.partial(
        pl.kernel,
        mesh=mesh,
        out_type=jax.ShapeDtypeStruct((N, H), jnp.float32),
        compiler_params=pltpu.CompilerParams(use_tc_tiling_on_sc=False),
        scratch_types=[
            pltpu.VMEM((ZCH, H), jnp.float32),
            [pltpu.VMEM((K,), jnp.int32) for _ in range(2)],
            [pltpu.VMEM((K,), jnp.int32) for _ in range(2)],
            [pltpu.VMEM((K,), jnp.int32) for _ in range(2)],
            [pltpu.VMEM((K, H), jnp.float32) for _ in range(2)],
            [pltpu.VMEM((K // 2, 2 * H), jnp.float32) for _ in range(2)],
            pltpu.VMEM_SHARED((ROWS, H), jnp.float32),
            [pltpu.SemaphoreType.DMA for _ in range(2)],
            [pltpu.SemaphoreType.DMA for _ in range(2)],
            [pltpu.SemaphoreType.DMA for _ in range(2)],
            [pltpu.SemaphoreType.DMA for _ in range(2)],
        ],
    )
    def scat(h_hbm, ef_hbm, ei_hbm, out_hbm, zbuf, srcs, dsts, idxs,
             hrs, efs, aggs, semi, semg, seme, sems):
        c = lax.axis_index("c")
        s = lax.axis_index("s")

        zero16 = jnp.zeros((_L,), jnp.float32)

        def zfill(i, carry):
            r = i // (H // _L)
            q = i % (H // _L)
            zbuf[r, pl.ds(q * _L, _L)] = zero16
            return carry

        lax.fori_loop(0, ZCH * (H // _L), zfill, 0)

        def zcopy(i, carry):
            pltpu.sync_copy(zbuf, aggs.at[pl.ds(s * TROWS + i * ZCH, ZCH)])
            return carry

        lax.fori_loop(0, ZIT, zcopy, 0)
        plsc.subcore_barrier()

        base_row = c * HALF
        ebase = s * EPT

        def issue_idx(ch, b):
            eb = ebase + ch * K
            pltpu.async_copy(ei_hbm.at[pl.ds(eb, K)], srcs[b], semi[b])
            pltpu.async_copy(ei_hbm.at[pl.ds(E + eb, K)], dsts[b], semi[b])

        def drain_idx(b):
            pltpu.make_async_copy(ei_hbm.at[pl.ds(0, K)], srcs[b],
                                  semi[b]).wait()
            pltpu.make_async_copy(ei_hbm.at[pl.ds(0, K)], dsts[b],
                                  semi[b]).wait()

        def compute_idx(dref, iref, n):
            def ixb(j, cr):
                d = dref[pl.ds(j * _L, _L)]
                loc = d - base_row
                ok = (loc >= 0) & (loc < HALF)
                iref[pl.ds(j * _L, _L)] = jnp.where(ok, loc, HALF)
                return cr
            lax.fori_loop(0, n // _L, ixb, 0)

        def compute_msg(href, eref, n):
            def row(rr, cr):
                for half in (0, 1):
                    r = 2 * rr + half
                    for q in range(H // _L):
                        sl = pl.ds(q * _L, _L)
                        esl = pl.ds(half * H + q * _L, _L)
                        href[r, sl] = jnp.maximum(href[r, sl] + eref[rr, esl],
                                                  0.0)
                return cr
            lax.fori_loop(0, n // 2, row, 0)

        issue_idx(0, 0)
        drain_idx(0)
        pltpu.async_copy(h_hbm.at[srcs[0]], hrs[0], semg[0])
        pltpu.async_copy(ef_hbm.at[pl.ds(ebase // 2, K // 2)], efs[0], seme[0])
        issue_idx(1, 1)

        def pair(j, carry):
            for b in (0, 1):
                ch = 2 * j + b
                nb = 1 - b

                compute_idx(dsts[b], idxs[b], K)

                @pl.when(ch + 1 < NCHP)
                def _():
                    @pl.when(ch >= 1)
                    def _():
                        pltpu.make_async_copy(hrs[nb], aggs.at[pl.ds(0, K)],
                                              sems[nb]).wait()
                    drain_idx(nb)
                    pltpu.async_copy(h_hbm.at[srcs[nb]], hrs[nb], semg[nb])
                    pltpu.async_copy(
                        ef_hbm.at[pl.ds((ebase + (ch + 1) * K) // 2, K // 2)],
                        efs[nb], seme[nb])

                pltpu.make_async_copy(h_hbm.at[pl.ds(0, K)], hrs[b],
                                      semg[b]).wait()
                pltpu.make_async_copy(ef_hbm.at[pl.ds(0, K // 2)], efs[b],
                                      seme[b]).wait()

                @pl.when(ch + 2 < NCHP)
                def _():
                    issue_idx(ch + 2, b)

                compute_msg(hrs[b], efs[b], K)
                pltpu.async_copy(hrs[b], aggs.at[idxs[b]], sems[b], add=True)
            return carry

        lax.fori_loop(0, NP, pair, 0)
        pltpu.make_async_copy(hrs[0], aggs.at[pl.ds(0, K)], sems[0]).wait()
        pltpu.make_async_copy(hrs[1], aggs.at[pl.ds(0, K)], sems[1]).wait()

        tb = ebase + NCHP * K
        pltpu.sync_copy(ei_hbm.at[pl.ds(tb, K)], srcs[0])
        cp = pltpu.async_copy(h_hbm.at[srcs[0]], hrs[0], semg[0])
        pltpu.sync_copy(ei_hbm.at[pl.ds(E + tb, K)], dsts[0])
        pltpu.sync_copy(ef_hbm.at[pl.ds(tb // 2, K // 2)], efs[0])
        compute_idx(dsts[0], idxs[0], K)
        cp.wait()
        compute_msg(hrs[0], efs[0], K)
        pltpu.sync_copy(hrs[0], aggs.at[idxs[0]], add=True)
        plsc.subcore_barrier()

        @pl.when(s < _NS - 1)
        def _():
            pltpu.sync_copy(
                aggs.at[pl.ds(s * WR, WR)],
                out_hbm.at[pl.ds(c * HALF + s * WR, WR)],
            )

        @pl.when(s == _NS - 1)
        def _():
            pltpu.sync_copy(
                aggs.at[pl.ds((_NS - 1) * WR, WR_LAST)],
                out_hbm.at[pl.ds(c * HALF + (_NS - 1) * WR, WR_LAST)],
            )

    return scat(h, ef, ei1d)


_NBLK = 1000
_EBLK = 6400


def _dot(a, b):
    return jnp.dot(a, b, preferred_element_type=jnp.float32)


def _node_encoder(x_in, batch2, te, w1, b1, w2, b2, tw, tb):
    N = x_in.shape[0]
    grid = N // _NBLK

    def body(xb, bb, teb, w1b, b1b, w2b, b2b, twb, tbb, out):
        pre = _dot(xb[...], w1b[...]) + b1b[...]
        hcur = _dot(_silu(pre), w2b[...]) + b2b[...]
        tf = _dot(_silu(teb[...]), twb[...]) + tbb[...]
        ids = lax.broadcasted_iota(jnp.int32, (_NBLK, 16), 1)
        oh = (bb[...] == ids).astype(jnp.float32)
        out[...] = hcur + jnp.dot(oh, tf, precision=lax.Precision.HIGHEST,
                                  preferred_element_type=jnp.float32)

    full = lambda shape: pl.BlockSpec(shape, lambda i: (0, 0))
    return pl.pallas_call(
        body,
        grid=(grid,),
        in_specs=[
            pl.BlockSpec((_NBLK, 7), lambda i: (i, 0)),
            pl.BlockSpec((_NBLK, 1), lambda i: (i, 0)),
            full((16, 32)),
            full((7, H)), full((1, H)),
            full((H, H)), full((1, H)),
            full((32, H)), full((1, H)),
        ],
        out_specs=pl.BlockSpec((_NBLK, H), lambda i: (i, 0)),
        out_shape=jax.ShapeDtypeStruct((N, H), jnp.float32),
    )(x_in, batch2, te, w1, b1, w2, b2, tw, tb)


def _edge_mlp(eaT, w1, b1, w2, b2):
    E = eaT.shape[1]
    grid = E // _EBLK

    def body(eb, w1b, b1b, w2b, b2b, out):
        pre = lax.dot_general(eb[...], w1b[...], (((0,), (0,)), ((), ())),
                              preferred_element_type=jnp.float32)
        hid = _silu(pre + b1b[...])
        out[...] = _dot(hid, w2b[...]) + b2b[...]

    full = lambda shape: pl.BlockSpec(shape, lambda i: (0, 0))
    return pl.pallas_call(
        body,
        grid=(grid,),
        in_specs=[
            pl.BlockSpec((2, _EBLK), lambda i: (0, i)),
            full((2, H)), full((1, H)), full((H, H)), full((1, H)),
        ],
        out_specs=pl.BlockSpec((_EBLK, H), lambda i: (i, 0)),
        out_shape=jax.ShapeDtypeStruct((E, H), jnp.float32),
    )(eaT, w1, b1, w2, b2)


def _layer_mlp(h, aggp, w1, b1, w2, b2):
    N = h.shape[0]
    grid = N // _NBLK

    def body(hb, ab, w1b, b1b, w2b, b2b, out):
        z = hb[...] + ab[...]
        hid = _silu(_dot(z, w1b[...]) + b1b[...])
        out[...] = _silu(_dot(hid, w2b[...]) + b2b[...])

    full = lambda shape: pl.BlockSpec(shape, lambda i: (0, 0))
    return pl.pallas_call(
        body,
        grid=(grid,),
        in_specs=[
            pl.BlockSpec((_NBLK, H), lambda i: (i, 0)),
            pl.BlockSpec((_NBLK, H), lambda i: (i, 0)),
            full((H, H)), full((1, H)), full((H, H)), full((1, H)),
        ],
        out_specs=pl.BlockSpec((_NBLK, H), lambda i: (i, 0)),
        out_shape=jax.ShapeDtypeStruct((N, H), jnp.float32),
    )(h, aggp, w1, b1, w2, b2)


def _final_mlp(h, aggp, w1, b1, w2, b2, fw1, fb1, fw2, fb2):
    N = h.shape[0]
    grid = N // _NBLK

    def body(hb, ab, w1b, b1b, w2b, b2b, fw1b, fb1b, fw2b, fb2b, out):
        z = hb[...] + ab[...]
        hid = _silu(_dot(z, w1b[...]) + b1b[...])
        h3 = _silu(_dot(hid, w2b[...]) + b2b[...])
        fh = _silu(_dot(h3, fw1b[...]) + fb1b[...])
        out[...] = _dot(fh, fw2b[...]) + fb2b[...]

    full = lambda shape: pl.BlockSpec(shape, lambda i: (0, 0))
    return pl.pallas_call(
        body,
        grid=(grid,),
        in_specs=[
            pl.BlockSpec((_NBLK, H), lambda i: (i, 0)),
            pl.BlockSpec((_NBLK, H), lambda i: (i, 0)),
            full((H, H)), full((1, H)), full((H, H)), full((1, H)),
            full((H, H)), full((1, H)), full((H, 1)), full((1, 1)),
        ],
        out_specs=pl.BlockSpec((_NBLK, 1), lambda i: (i, 0)),
        out_shape=jax.ShapeDtypeStruct((N, 1), jnp.float32),
    )(h, aggp, w1, b1, w2, b2, fw1, fb1, fw2, fb2)


def kernel(x_t, t, condition, edge_index, edge_attr, batch, params):
    N = x_t.shape[0]
    E = edge_index.shape[1]
    p = params

    half = 16
    freq = jnp.exp(jnp.arange(half, dtype=jnp.float32)
                   * (-math.log(10000.0) / (half - 1)))
    ang = t[:, None].astype(jnp.float32) * freq[None, :]
    te = jnp.concatenate([jnp.sin(ang), jnp.cos(ang)], axis=-1)

    r1 = lambda v: v.reshape(1, -1)
    batch2 = batch.reshape(N, 1)

    x_in = jnp.concatenate([x_t, condition], axis=-1)
    h = _node_encoder(
        x_in, batch2, te,
        p['node_w1'], r1(p['node_b1']),
        p['node_w2'], r1(p['node_b2']),
        p['time_w'], r1(p['time_b']),
    )
    ef = _edge_mlp(edge_attr.T, p['edge_w1'], r1(p['edge_b1']),
                   p['edge_w2'], r1(p['edge_b2'])).reshape(E // 2, 2 * H)

    ei1d = edge_index.reshape(2 * E)
    for name in ('c1', 'c2'):
        aggp = _sc_scatter(h, ef, ei1d, N=N, E=E)
        h = _layer_mlp(h, aggp, p[name + '_w1'], r1(p[name + '_b1']),
                       p[name + '_w2'], r1(p[name + '_b2']))

    aggp = _sc_scatter(h, ef, ei1d, N=N, E=E)
    return _final_mlp(h, aggp, p['c3_w1'], r1(p['c3_b1']),
                      p['c3_w2'], r1(p['c3_b2']),
                      p['f_w1'], r1(p['f_b1']),
                      p['f_w2'], jnp.full((1, 1), p['f_b2'][0]))
